# baseline (speedup 1.0000x reference)
import functools

import jax
import jax.numpy as jnp
from jax import lax
from jax.experimental import pallas as pl
from jax.experimental.pallas import tpu as pltpu
from jax.experimental.pallas import tpu_sc as plsc

N = 10000
E = 320000
D = 128
G = 64
FP = 512

NC = 2
NS = 16
NW = NC * NS
B = 80
CH = E // (NW * B)
W = CH
TR = 632
TRL = N - (NS - 1) * TR

BLK = 2000
NBLK = N // BLK
PW = 512


@functools.cache
def _sc_mesh():
    return plsc.VectorSubcoreMesh(
        core_axis_name="c", subcore_axis_name="s", num_cores=NC, num_subcores=NS)


def _zero(zeros_hbm, dst, s):
    base = jnp.minimum(s * TR, N - TR)
    pltpu.sync_copy(zeros_hbm, dst.at[pl.ds(base, TR)])


def _staged(src, dst, s):
    base = jnp.minimum(s * TR, N - TR)
    pltpu.sync_copy(src.at[pl.ds(base, TR)], dst.at[pl.ds(base, TR)])


def _deg_body(dst3, ones_hbm, zeros_hbm, out, dst_all, ones_v, acc):
    c = lax.axis_index("c")
    s = lax.axis_index("s")
    w = c * NS + s
    _zero(zeros_hbm, acc, s)
    pltpu.sync_copy(ones_hbm, ones_v)
    pltpu.sync_copy(dst3.at[w], dst_all)
    plsc.subcore_barrier()

    def chunk(i, carry):
        pltpu.sync_copy(ones_v, acc.at[dst_all.at[i]], add=True)
        return carry

    lax.fori_loop(0, CH, chunk, 0)
    plsc.subcore_barrier()
    _staged(acc, out.at[c], s)


@functools.cache
def _deg_kernel():
    return pl.kernel(
        _deg_body,
        out_type=jax.ShapeDtypeStruct((NC, N, D), jnp.float32),
        mesh=_sc_mesh(),
        scratch_types=[
            pltpu.VMEM((CH, B), jnp.int32),
            pltpu.VMEM((B, D), jnp.float32),
            pltpu.VMEM_SHARED((N, D), jnp.float32),
        ],
    )


def _agg_body(table, srcf, dst3, zeros_hbm, out,
              src_flat, dst_w, rows, gsems, ssems, acc):
    c = lax.axis_index("c")
    s = lax.axis_index("s")
    w = c * NS + s
    _zero(zeros_hbm, acc, s)
    pltpu.sync_copy(srcf.at[w], src_flat)
    pltpu.sync_copy(dst3.at[w], dst_w)
    plsc.subcore_barrier()

    def gath(i, j):
        pltpu.async_copy(table.at[src_flat.at[pl.ds(i * B, B)]],
                         rows.at[j], gsems[j])

    def gwait(i, j):
        pltpu.make_async_copy(table.at[src_flat.at[pl.ds(i * B, B)]],
                              rows.at[j], gsems[j]).wait()

    def scat(i, j):
        pltpu.async_copy(rows.at[j], acc.at[dst_w.at[i]], ssems[j], add=True)

    def swait(i, j):
        pltpu.make_async_copy(rows.at[j], acc.at[dst_w.at[i]], ssems[j]).wait()

    def ssync(i, j):
        scat(i, j)
        swait(i, j)

    gath(0, 0)

    def pair(k, carry):
        i = 2 * k
        gath(i + 1, 1)
        gwait(i, 0)
        ssync(i, 0)
        gath(i + 2, 0)
        gwait(i + 1, 1)
        ssync(i + 1, 1)
        return carry

    lax.fori_loop(0, (W - 1) // 2, pair, 0)
    gwait(W - 1, 0)
    ssync(W - 1, 0)

    plsc.subcore_barrier()
    _staged(acc, out.at[c], s)


@functools.cache
def _agg_kernel():
    return pl.kernel(
        _agg_body,
        out_type=jax.ShapeDtypeStruct((NC, N, D), jnp.float32),
        mesh=_sc_mesh(),
        scratch_types=[
            pltpu.VMEM((W * B,), jnp.int32),
            pltpu.VMEM((W, B), jnp.int32),
            pltpu.VMEM((2, B, D), jnp.float32),
            [pltpu.SemaphoreType.DMA] * 2,
            [pltpu.SemaphoreType.DMA] * 2,
            pltpu.VMEM_SHARED((N, D), jnp.float32),
        ],
    )


def _dinv(deg2_ref):
    deg = deg2_ref[0, :, :1] + deg2_ref[1, :, :1] + 1.0
    return lax.rsqrt(deg)


def _conv1_body(x_ref, w1_ref, deg2_ref, hs1_ref, dinv_ref):
    dinv = _dinv(deg2_ref)
    h = jnp.dot(x_ref[...], w1_ref[...], preferred_element_type=jnp.float32)
    hs1_ref[...] = h * dinv
    dinv_ref[...] = jnp.broadcast_to(dinv, (BLK, 8))


def _conv2_body(agg_ref, hs1_ref, dinv8_ref, b1_ref, w2_ref, hs2_ref):
    dinv = dinv8_ref[:, :1]
    h1 = dinv * (agg_ref[0] + agg_ref[1] + hs1_ref[...]) + b1_ref[...]
    h1 = jnp.maximum(h1, 0.0)
    hs2_ref[...] = jnp.dot(h1, w2_ref[...], preferred_element_type=jnp.float32) * dinv


def _pool_body(agg_ref, hs2_ref, dinv8_ref, b2_ref, ww_ref, bw_ref,
               bc_ref, wout_ref, bout_ref, out_ref, wsum_ref, pmax_ref):
    i = pl.program_id(0)

    @pl.when(i == 0)
    def _init():
        wsum_ref[...] = jnp.zeros_like(wsum_ref)
        pmax_ref[...] = jnp.full_like(pmax_ref, -jnp.inf)

    dinv = dinv8_ref[:, :1]
    h2 = dinv * (agg_ref[0] + agg_ref[1] + hs2_ref[...]) + b2_ref[...]
    h2 = jnp.maximum(h2, 0.0)
    wgt = jax.nn.sigmoid(
        jnp.dot(h2, ww_ref[...], preferred_element_type=jnp.float32) + bw_ref[0, 0])
    bc = bc_ref[...]
    giota = lax.broadcasted_iota(jnp.int32, (1, G), 1).astype(jnp.float32)
    onehot = (bc == giota).astype(jnp.float32)
    wsum_ref[...] += lax.dot_general(
        onehot, h2 * wgt, (((0,), (0,)), ((), ())),
        preferred_element_type=jnp.float32)

    gmin = jnp.min(bc).astype(jnp.int32)
    gmax = jnp.max(bc).astype(jnp.int32)

    def seg(g, carry):
        m = bc == g.astype(jnp.float32)
        cm = jnp.max(jnp.where(m, h2, -jnp.inf), axis=0, keepdims=True)
        pmax_ref[pl.ds(g, 1), :] = jnp.maximum(pmax_ref[pl.ds(g, 1), :], cm)
        return carry

    lax.fori_loop(gmin, gmax + 1, seg, 0)

    @pl.when(i == NBLK - 1)
    def _final():
        acc = jnp.dot(wsum_ref[...], wout_ref[:D], preferred_element_type=jnp.float32)
        acc += jnp.dot(pmax_ref[...], wout_ref[D:], preferred_element_type=jnp.float32)
        out_ref[...] = jnp.maximum(acc + bout_ref[...], 0.0)


def _row_block(i):
    return (i, 0)


def _row_block3(i):
    return (0, i, 0)


def _whole(i):
    return (0, 0)


_deg_spec = pl.BlockSpec((NC, BLK, D), _row_block3)
_agg_spec = pl.BlockSpec((NC, BLK, D), _row_block3)

_conv1 = pl.pallas_call(
    _conv1_body,
    grid=(NBLK,),
    in_specs=[
        pl.BlockSpec((BLK, D), _row_block),
        pl.BlockSpec((D, D), _whole),
        _deg_spec,
    ],
    out_specs=[
        pl.BlockSpec((BLK, D), _row_block),
        pl.BlockSpec((BLK, 8), _row_block),
    ],
    out_shape=[
        jax.ShapeDtypeStruct((N, D), jnp.float32),
        jax.ShapeDtypeStruct((N, 8), jnp.float32),
    ],
)

_conv2 = pl.pallas_call(
    _conv2_body,
    grid=(NBLK,),
    in_specs=[
        _agg_spec,
        pl.BlockSpec((BLK, D), _row_block),
        pl.BlockSpec((BLK, 8), _row_block),
        pl.BlockSpec((1, D), _whole),
        pl.BlockSpec((D, D), _whole),
    ],
    out_specs=pl.BlockSpec((BLK, D), _row_block),
    out_shape=jax.ShapeDtypeStruct((N, D), jnp.float32),
)

_pool = pl.pallas_call(
    _pool_body,
    grid=(NBLK,),
    in_specs=[
        _agg_spec,
        pl.BlockSpec((BLK, D), _row_block),
        pl.BlockSpec((BLK, 8), _row_block),
        pl.BlockSpec((1, D), _whole),
        pl.BlockSpec((D, 1), _whole),
        pl.BlockSpec((1, 1), _whole),
        pl.BlockSpec((BLK, 1), _row_block),
        pl.BlockSpec((2 * D, FP), _whole),
        pl.BlockSpec((1, FP), _whole),
    ],
    out_specs=pl.BlockSpec((G, FP), _whole),
    out_shape=jax.ShapeDtypeStruct((G, FP), jnp.float32),
    scratch_shapes=[
        pltpu.VMEM((G, D), jnp.float32),
        pltpu.VMEM((G, D), jnp.float32),
    ],
)


def kernel(feats, edge_index, batch, W1, b1, W2, b2, Ww, bw, Wout, bout):
    dst3 = edge_index[1].reshape(NW, W, B)
    srcf = edge_index[0].reshape(NW, W * B)
    batch_col = batch.astype(jnp.float32).reshape(N, 1)
    zeros128 = jnp.zeros((TR, D), jnp.float32)
    ones128 = jnp.ones((B, D), jnp.float32)
    b1r = b1.reshape(1, D)
    b2r = b2.reshape(1, D)
    bwr = bw.reshape(1, 1)
    boutr = bout.reshape(1, FP)

    deg2 = _deg_kernel()(dst3, ones128, zeros128)
    hs1, dinv8 = _conv1(feats, W1, deg2)
    agg1 = _agg_kernel()(hs1, srcf, dst3, zeros128)
    hs2 = _conv2(agg1, hs1, dinv8, b1r, W2)
    agg2 = _agg_kernel()(hs2, srcf, dst3, zeros128)
    out = _pool(agg2, hs2, dinv8, b2r, Ww, bwr, batch_col, Wout, boutr)
    return out


# speedup vs baseline: 1.0378x; 1.0378x over previous
import functools

import jax
import jax.numpy as jnp
from jax import lax
from jax.experimental import pallas as pl
from jax.experimental.pallas import tpu as pltpu
from jax.experimental.pallas import tpu_sc as plsc

N = 10000
E = 320000
D = 128
G = 64
FP = 512

NC = 2
NS = 16
NW = NC * NS
B = 80
CH = E // (NW * B)
W = CH
TR = 632
TRL = N - (NS - 1) * TR

BLK = 2000
NBLK = N // BLK
PW = 512


@functools.cache
def _sc_mesh():
    return plsc.VectorSubcoreMesh(
        core_axis_name="c", subcore_axis_name="s", num_cores=NC, num_subcores=NS)


def _zero(zeros_hbm, dst, s):
    base = jnp.minimum(s * TR, N - TR)
    pltpu.sync_copy(zeros_hbm, dst.at[pl.ds(base, TR)])


def _staged(src, dst, s):
    base = jnp.minimum(s * TR, N - TR)
    pltpu.sync_copy(src.at[pl.ds(base, TR)], dst.at[pl.ds(base, TR)])


def _deg_body(edges, ones_hbm, zeros_hbm, out, dst_all, ones_v, acc):
    c = lax.axis_index("c")
    s = lax.axis_index("s")
    w = c * NS + s
    _zero(zeros_hbm, acc, s)
    pltpu.sync_copy(ones_hbm, ones_v)
    pltpu.sync_copy(edges.at[1].at[w], dst_all)
    plsc.subcore_barrier()

    def chunk(i, carry):
        pltpu.sync_copy(ones_v, acc.at[dst_all.at[i]], add=True)
        return carry

    lax.fori_loop(0, CH, chunk, 0)
    plsc.subcore_barrier()
    _staged(acc, out.at[c], s)


@functools.cache
def _deg_kernel():
    return pl.kernel(
        _deg_body,
        out_type=jax.ShapeDtypeStruct((NC, N, D), jnp.float32),
        mesh=_sc_mesh(),
        scratch_types=[
            pltpu.VMEM((CH, B), jnp.int32),
            pltpu.VMEM((B, D), jnp.float32),
            pltpu.VMEM_SHARED((N, D), jnp.float32),
        ],
    )


def _agg_body(table, srcf, edges, zeros_hbm, out,
              src_flat, dst_w, rows, gsems, ssems, acc):
    c = lax.axis_index("c")
    s = lax.axis_index("s")
    w = c * NS + s
    _zero(zeros_hbm, acc, s)
    pltpu.sync_copy(srcf.at[w], src_flat)
    pltpu.sync_copy(edges.at[1].at[w], dst_w)
    plsc.subcore_barrier()

    def gath(i, j):
        pltpu.async_copy(table.at[src_flat.at[pl.ds(i * B, B)]],
                         rows.at[j], gsems[j])

    def gwait(i, j):
        pltpu.make_async_copy(table.at[src_flat.at[pl.ds(i * B, B)]],
                              rows.at[j], gsems[j]).wait()

    def scat(i, j):
        pltpu.async_copy(rows.at[j], acc.at[dst_w.at[i]], ssems[j], add=True)

    def swait(i, j):
        pltpu.make_async_copy(rows.at[j], acc.at[dst_w.at[i]], ssems[j]).wait()

    def ssync(i, j):
        scat(i, j)
        swait(i, j)

    gath(0, 0)

    def pair(k, carry):
        i = 2 * k
        gath(i + 1, 1)
        gwait(i, 0)
        ssync(i, 0)
        gath(i + 2, 0)
        gwait(i + 1, 1)
        ssync(i + 1, 1)
        return carry

    lax.fori_loop(0, (W - 1) // 2, pair, 0)
    gwait(W - 1, 0)
    ssync(W - 1, 0)

    plsc.subcore_barrier()
    _staged(acc, out.at[c], s)


@functools.cache
def _agg_kernel():
    return pl.kernel(
        _agg_body,
        out_type=jax.ShapeDtypeStruct((NC, N, D), jnp.float32),
        mesh=_sc_mesh(),
        scratch_types=[
            pltpu.VMEM((W * B,), jnp.int32),
            pltpu.VMEM((W, B), jnp.int32),
            pltpu.VMEM((2, B, D), jnp.float32),
            [pltpu.SemaphoreType.DMA] * 2,
            [pltpu.SemaphoreType.DMA] * 2,
            pltpu.VMEM_SHARED((N, D), jnp.float32),
        ],
    )


def _dinv(deg2_ref):
    deg = deg2_ref[0, :, :1] + deg2_ref[1, :, :1] + 1.0
    return lax.rsqrt(deg)


def _conv1_body(x_ref, w1_ref, deg2_ref, hs1_ref, dinv_ref):
    dinv = _dinv(deg2_ref)
    h = jnp.dot(x_ref[...], w1_ref[...], preferred_element_type=jnp.float32)
    hs1_ref[...] = h * dinv
    dinv_ref[...] = jnp.broadcast_to(dinv, (BLK, 8))


def _conv2_body(agg_ref, hs1_ref, dinv8_ref, b1_ref, w2_ref, hs2_ref):
    dinv = dinv8_ref[:, :1]
    h1 = dinv * (agg_ref[0] + agg_ref[1] + hs1_ref[...]) + b1_ref[...]
    h1 = jnp.maximum(h1, 0.0)
    hs2_ref[...] = jnp.dot(h1, w2_ref[...], preferred_element_type=jnp.float32) * dinv


def _pool_body(agg_ref, hs2_ref, dinv8_ref, b2_ref, ww_ref, bw_ref,
               bc_ref, wout_ref, bout_ref, out_ref, wsum_ref, pmax_ref,
               h2s_ref, stats_ref):
    i = pl.program_id(0)

    @pl.when(i == 0)
    def _init():
        wsum_ref[...] = jnp.zeros_like(wsum_ref)
        pmax_ref[...] = jnp.full_like(pmax_ref, -jnp.inf)

    dinv = dinv8_ref[:, :1]
    h2 = dinv * (agg_ref[0] + agg_ref[1] + hs2_ref[...]) + b2_ref[...]
    h2 = jnp.maximum(h2, 0.0)
    wgt = jax.nn.sigmoid(
        jnp.dot(h2, ww_ref[...], preferred_element_type=jnp.float32) + bw_ref[0, 0])
    bc = bc_ref[...]
    giota = lax.broadcasted_iota(jnp.int32, (1, G), 1).astype(jnp.float32)
    onehot = (bc == giota).astype(jnp.float32)
    wsum_ref[...] += lax.dot_general(
        onehot, h2 * wgt, (((0,), (0,)), ((), ())),
        preferred_element_type=jnp.float32)

    gmin = jnp.min(bc).astype(jnp.int32)
    gmax = jnp.max(bc).astype(jnp.int32)

    onesc = jnp.ones((BLK, 1), jnp.float32)
    cnt = lax.dot_general(onehot, onesc, (((0,), (0,)), ((), ())),
                          preferred_element_type=jnp.float32)
    sgt = (lax.broadcasted_iota(jnp.int32, (G, G), 1)
           < lax.broadcasted_iota(jnp.int32, (G, G), 0)).astype(jnp.float32)
    starts = jnp.dot(sgt, cnt, preferred_element_type=jnp.float32)
    h2s_ref[...] = h2
    stats_ref[:, 0:1] = cnt
    stats_ref[:, 1:2] = starts

    def seg(g, carry):
        gf = g.astype(jnp.float32)
        c_g = stats_ref[g, 0]
        st = jnp.minimum(stats_ref[g, 1].astype(jnp.int32), BLK - PW)

        @pl.when(c_g <= float(PW))
        def _small():
            m = bc_ref[pl.ds(st, PW), :] == gf
            cm = jnp.max(jnp.where(m, h2s_ref[pl.ds(st, PW), :], -jnp.inf),
                         axis=0, keepdims=True)
            pmax_ref[pl.ds(g, 1), :] = jnp.maximum(pmax_ref[pl.ds(g, 1), :], cm)

        @pl.when(c_g > float(PW))
        def _big():
            m = bc_ref[...] == gf
            cm = jnp.max(jnp.where(m, h2s_ref[...], -jnp.inf),
                         axis=0, keepdims=True)
            pmax_ref[pl.ds(g, 1), :] = jnp.maximum(pmax_ref[pl.ds(g, 1), :], cm)

        return carry

    lax.fori_loop(gmin, gmax + 1, seg, 0)

    @pl.when(i == NBLK - 1)
    def _final():
        acc = jnp.dot(wsum_ref[...], wout_ref[:D], preferred_element_type=jnp.float32)
        acc += jnp.dot(pmax_ref[...], wout_ref[D:], preferred_element_type=jnp.float32)
        out_ref[...] = jnp.maximum(acc + bout_ref[...], 0.0)


def _row_block(i):
    return (i, 0)


def _row_block3(i):
    return (0, i, 0)


def _whole(i):
    return (0, 0)


_deg_spec = pl.BlockSpec((NC, BLK, D), _row_block3)
_agg_spec = pl.BlockSpec((NC, BLK, D), _row_block3)

_conv1 = pl.pallas_call(
    _conv1_body,
    grid=(NBLK,),
    in_specs=[
        pl.BlockSpec((BLK, D), _row_block),
        pl.BlockSpec((D, D), _whole),
        _deg_spec,
    ],
    out_specs=[
        pl.BlockSpec((BLK, D), _row_block),
        pl.BlockSpec((BLK, 8), _row_block),
    ],
    out_shape=[
        jax.ShapeDtypeStruct((N, D), jnp.float32),
        jax.ShapeDtypeStruct((N, 8), jnp.float32),
    ],
)

_conv2 = pl.pallas_call(
    _conv2_body,
    grid=(NBLK,),
    in_specs=[
        _agg_spec,
        pl.BlockSpec((BLK, D), _row_block),
        pl.BlockSpec((BLK, 8), _row_block),
        pl.BlockSpec((1, D), _whole),
        pl.BlockSpec((D, D), _whole),
    ],
    out_specs=pl.BlockSpec((BLK, D), _row_block),
    out_shape=jax.ShapeDtypeStruct((N, D), jnp.float32),
)

_pool = pl.pallas_call(
    _pool_body,
    grid=(NBLK,),
    in_specs=[
        _agg_spec,
        pl.BlockSpec((BLK, D), _row_block),
        pl.BlockSpec((BLK, 8), _row_block),
        pl.BlockSpec((1, D), _whole),
        pl.BlockSpec((D, 1), _whole),
        pl.BlockSpec((1, 1), _whole),
        pl.BlockSpec((BLK, 1), _row_block),
        pl.BlockSpec((2 * D, FP), _whole),
        pl.BlockSpec((1, FP), _whole),
    ],
    out_specs=pl.BlockSpec((G, FP), _whole),
    out_shape=jax.ShapeDtypeStruct((G, FP), jnp.float32),
    scratch_shapes=[
        pltpu.VMEM((G, D), jnp.float32),
        pltpu.VMEM((G, D), jnp.float32),
        pltpu.VMEM((BLK, D), jnp.float32),
        pltpu.VMEM((G, 8), jnp.float32),
    ],
)


def kernel(feats, edge_index, batch, W1, b1, W2, b2, Ww, bw, Wout, bout):
    edges = edge_index.reshape(2, NW, W, B)
    srcf = edge_index[0].reshape(NW, W * B)
    batch_col = batch.astype(jnp.float32).reshape(N, 1)
    zeros128 = jnp.zeros((TR, D), jnp.float32)
    ones128 = jnp.ones((B, D), jnp.float32)
    b1r = b1.reshape(1, D)
    b2r = b2.reshape(1, D)
    bwr = bw.reshape(1, 1)
    boutr = bout.reshape(1, FP)

    deg2 = _deg_kernel()(edges, ones128, zeros128)
    hs1, dinv8 = _conv1(feats, W1, deg2)
    agg1 = _agg_kernel()(hs1, srcf, edges, zeros128)
    hs2 = _conv2(agg1, hs1, dinv8, b1r, W2)
    agg2 = _agg_kernel()(hs2, srcf, edges, zeros128)
    out = _pool(agg2, hs2, dinv8, b2r, Ww, bwr, batch_col, Wout, boutr)
    return out
